# baseline (speedup 1.0000x reference)
import jax
import jax.numpy as jnp
from jax import lax
from jax.experimental import pallas as pl
from jax.experimental.pallas import tpu as pltpu
from jax.experimental.pallas import tpu_sc as plsc

TOD, DOW, DOM, DOY = 288, 7, 31, 366
D = 64
B = 4096
S = 200
NC, NS = 2, 16
NW = NC * NS
NB = B // NW
R_DD = DOW * DOM
ROWS = TOD + R_DD + DOY
SP = NB + 1


def _sc_body(x_hbm, tab_hbm, out_hbm, tab_v, x_v, out_v, sx, so):
    wid = lax.axis_index("s") * NC + lax.axis_index("c")
    b0 = wid * NB
    pltpu.sync_copy(tab_hbm, tab_v)

    iota = lax.iota(jnp.int32, 16)
    out_v2 = [ov.at[:, pl.ds(0, NB)] for ov in out_v]

    pltpu.async_copy(x_hbm.at[0, :, pl.ds(b0, NB)], x_v[0], sx[0])

    def compute_chunk(x_vp, out_vp):
        @plsc.parallel_loop(0, NB // 16, unroll=2)
        def group_body(g):
            xv0 = x_vp[0, pl.ds(16 * g, 16)]
            xv1 = x_vp[1, pl.ds(16 * g, 16)]
            xv2 = x_vp[2, pl.ds(16 * g, 16)]
            xv3 = x_vp[3, pl.ds(16 * g, 16)]
            rv0 = ((xv0 + 0.5) * float(TOD)).astype(jnp.int32) * D
            i1 = ((xv1 + 0.5) * float(DOW)).astype(jnp.int32)
            i2 = ((xv2 + 0.5) * float(DOM)).astype(jnp.int32)
            rv1 = i1 * (DOM * D) + i2 * D + TOD * D
            rv2 = (((xv3 + 0.5) * float(DOY)).astype(jnp.int32) * D
                   + (TOD + R_DD) * D)
            for k in range(16):
                r0 = rv0[k]
                r1 = rv1[k]
                r2 = rv2[k]
                tvec = jnp.zeros((16,), jnp.int32) + (16 * g + k)
                for c in range(0, D, 16):
                    v = (tab_v[pl.ds(r0 + c, 16)]
                         + tab_v[pl.ds(r1 + c, 16)]
                         + tab_v[pl.ds(r2 + c, 16)])
                    plsc.store_scatter(out_vp, [iota + c, tvec], v)

    def s_pair(sj, carry):
        for p in range(2):
            s = 2 * sj + p
            pltpu.make_async_copy(
                x_hbm.at[s, :, pl.ds(b0, NB)], x_v[p], sx[p]).wait()

            @pl.when(s + 1 < S)
            def _():
                pltpu.async_copy(
                    x_hbm.at[s + 1, :, pl.ds(b0, NB)], x_v[1 - p], sx[1 - p])

            @pl.when(s >= 2)
            def _():
                pltpu.make_async_copy(
                    out_v2[p], out_hbm.at[s - 2, :, pl.ds(b0, NB)],
                    so[p]).wait()

            compute_chunk(x_v[p], out_v[p])
            pltpu.async_copy(
                out_v2[p], out_hbm.at[s, :, pl.ds(b0, NB)], so[p])
        return carry

    lax.fori_loop(0, S // 2, s_pair, 0)
    for p in range(2):
        pltpu.make_async_copy(
            out_v2[p], out_hbm.at[S - 2 + p, :, pl.ds(b0, NB)], so[p]).wait()


def kernel(x, w_tod, w_dow, w_dom, w_doy):
    w_dd = (w_dow[:, None, :] + w_dom[None, :, :]).reshape(R_DD, D)
    tab = jnp.concatenate([w_tod, w_dd, w_doy], axis=0).reshape(-1)
    xt = x.transpose(1, 2, 0)
    mesh = plsc.VectorSubcoreMesh(core_axis_name="c", subcore_axis_name="s")
    out = pl.kernel(
        _sc_body,
        out_type=jax.ShapeDtypeStruct((S, D, B), jnp.float32),
        mesh=mesh,
        scratch_types=[
            pltpu.VMEM((ROWS * D,), jnp.float32),
            [pltpu.VMEM((4, NB), jnp.float32)] * 2,
            [pltpu.VMEM((D, SP), jnp.float32)] * 2,
            [pltpu.SemaphoreType.DMA] * 2,
            [pltpu.SemaphoreType.DMA] * 2,
        ],
        compiler_params=pltpu.CompilerParams(needs_layout_passes=False),
    )(xt, tab)
    return out.transpose(2, 0, 1)


# speedup vs baseline: 2.6388x; 2.6388x over previous
import jax
import jax.numpy as jnp
from jax import lax
from jax.experimental import pallas as pl
from jax.experimental.pallas import tpu as pltpu
from jax.experimental.pallas import tpu_sc as plsc

TOD, DOW, DOM, DOY = 288, 7, 31, 366
D = 64
B = 4096
S = 200
NC, NS = 2, 16
NW = NC * NS
NB = B // NW
R_DD = DOW * DOM
ROWS = TOD + R_DD + DOY
SP = D + 1


def _sc_body(x_hbm, tab_hbm, out_hbm, tab_v, x_v, out_v, sx, so):
    wid = lax.axis_index("s") * NC + lax.axis_index("c")
    b0 = wid * NB
    pltpu.sync_copy(tab_hbm, tab_v)

    out_v2 = out_v

    pltpu.async_copy(x_hbm.at[0, :, pl.ds(b0, NB)], x_v[0], sx[0])

    def compute_chunk(x_vp, out_vp):
        @plsc.parallel_loop(0, NB // 16, unroll=2)
        def group_body(g):
            xv0 = x_vp[0, pl.ds(16 * g, 16)]
            xv1 = x_vp[1, pl.ds(16 * g, 16)]
            xv2 = x_vp[2, pl.ds(16 * g, 16)]
            xv3 = x_vp[3, pl.ds(16 * g, 16)]
            rv0 = ((xv0 + 0.5) * float(TOD)).astype(jnp.int32) * SP
            i1 = ((xv1 + 0.5) * float(DOW)).astype(jnp.int32)
            i2 = ((xv2 + 0.5) * float(DOM)).astype(jnp.int32)
            rv1 = i1 * (DOM * SP) + i2 * SP + TOD * SP
            rv2 = (((xv3 + 0.5) * float(DOY)).astype(jnp.int32) * SP
                   + (TOD + R_DD) * SP)
            for d in range(D):
                v = (plsc.load_gather(tab_v, [rv0 + d])
                     + plsc.load_gather(tab_v, [rv1 + d])
                     + plsc.load_gather(tab_v, [rv2 + d]))
                out_vp[d, pl.ds(16 * g, 16)] = v

    def s_pair(sj, carry):
        for p in range(2):
            s = 2 * sj + p
            pltpu.make_async_copy(
                x_hbm.at[s, :, pl.ds(b0, NB)], x_v[p], sx[p]).wait()

            @pl.when(s + 1 < S)
            def _():
                pltpu.async_copy(
                    x_hbm.at[s + 1, :, pl.ds(b0, NB)], x_v[1 - p], sx[1 - p])

            @pl.when(s >= 2)
            def _():
                pltpu.make_async_copy(
                    out_v2[p], out_hbm.at[s - 2, :, pl.ds(b0, NB)],
                    so[p]).wait()

            compute_chunk(x_v[p], out_v[p])
            pltpu.async_copy(
                out_v2[p], out_hbm.at[s, :, pl.ds(b0, NB)], so[p])
        return carry

    lax.fori_loop(0, S // 2, s_pair, 0)
    for p in range(2):
        pltpu.make_async_copy(
            out_v2[p], out_hbm.at[S - 2 + p, :, pl.ds(b0, NB)], so[p]).wait()


def kernel(x, w_tod, w_dow, w_dom, w_doy):
    w_dd = (w_dow[:, None, :] + w_dom[None, :, :]).reshape(R_DD, D)
    tab = jnp.concatenate([w_tod, w_dd, w_doy], axis=0)
    tab = jnp.pad(tab, ((0, 0), (0, SP - D))).reshape(-1)
    xt = x.transpose(1, 2, 0)
    mesh = plsc.VectorSubcoreMesh(core_axis_name="c", subcore_axis_name="s")
    out = pl.kernel(
        _sc_body,
        out_type=jax.ShapeDtypeStruct((S, D, B), jnp.float32),
        mesh=mesh,
        scratch_types=[
            pltpu.VMEM((ROWS * SP,), jnp.float32),
            [pltpu.VMEM((4, NB), jnp.float32)] * 2,
            [pltpu.VMEM((D, NB), jnp.float32)] * 2,
            [pltpu.SemaphoreType.DMA] * 2,
            [pltpu.SemaphoreType.DMA] * 2,
        ],
        compiler_params=pltpu.CompilerParams(needs_layout_passes=False),
    )(xt, tab)
    return out.transpose(2, 0, 1)
